# baseline (speedup 1.0000x reference)
import jax
import jax.numpy as jnp
from jax.experimental import pallas as pl
from jax.experimental.pallas import tpu as pltpu

_PAIRS = [(0, 1), (0, 2), (0, 3), (1, 2), (1, 3), (2, 3)]


def _edm_body(x1_hbm, x2_hbm, x3_hbm, x4_hbm, out_ref,
              b1, b2, b3, b4, sems):
    hbms = (x1_hbm, x2_hbm, x3_hbm, x4_hbm)
    bufs = (b1, b2, b3, b4)
    bn = b1.shape[1]
    i = pl.program_id(0)
    grid = pl.num_programs(0)
    slot = jax.lax.rem(i, 2)
    nxt = 1 - slot

    def start_copies(step, buf_slot):
        for t in range(4):
            pltpu.make_async_copy(
                hbms[t].at[pl.ds(step * bn, bn), :],
                bufs[t].at[buf_slot],
                sems.at[t, buf_slot],
            ).start(priority=t % 2)

    @pl.when(i == 0)
    def _():
        start_copies(0, 0)

    @pl.when(i + 1 < grid)
    def _():
        start_copies(i + 1, nxt)

    es = []
    n2 = []
    for t in range(4):
        pltpu.make_async_copy(
            hbms[t].at[pl.ds(i * bn, bn), :],
            bufs[t].at[slot],
            sems.at[t, slot],
        ).wait()
        x = bufs[t][slot]
        m = jnp.max(x, axis=-1, keepdims=True)
        e = jnp.exp(x - m)
        es.append(e)
        n2.append(jnp.sum(e * e, axis=-1))
    sims = []
    for j, k in _PAIRS:
        d = jnp.sum(es[j] * es[k], axis=-1)
        sims.append(d * jax.lax.rsqrt(n2[j] * n2[k]))
    smax = sims[0]
    for s in sims[1:]:
        smax = jnp.maximum(smax, s)
    acc = jnp.zeros_like(smax)
    for s in sims:
        acc = acc + jnp.exp(s - smax)
    loss = jnp.log(acc) + smax
    out_ref[...] = jnp.sum(loss).reshape(1, 1, 1)


def kernel(outputs1, outputs2, outputs3, outputs4):
    n, c = outputs1.shape
    bn = 512
    grid = n // bn
    any_spec = pl.BlockSpec(memory_space=pl.ANY)
    partials = pl.pallas_call(
        _edm_body,
        grid=(grid,),
        in_specs=[any_spec, any_spec, any_spec, any_spec],
        out_specs=pl.BlockSpec((1, 1, 1), lambda i: (i, 0, 0)),
        out_shape=jax.ShapeDtypeStruct((grid, 1, 1), jnp.float32),
        scratch_shapes=[
            pltpu.VMEM((2, bn, c), jnp.float32),
            pltpu.VMEM((2, bn, c), jnp.float32),
            pltpu.VMEM((2, bn, c), jnp.float32),
            pltpu.VMEM((2, bn, c), jnp.float32),
            pltpu.SemaphoreType.DMA((4, 2)),
        ],
        compiler_params=pltpu.CompilerParams(
            dimension_semantics=("arbitrary",),
        ),
    )(outputs1, outputs2, outputs3, outputs4)
    return jnp.sum(partials) / n
